# baseline (speedup 1.0000x reference)
import functools

import jax
import jax.numpy as jnp
from jax import lax
from jax.experimental import pallas as pl
from jax.experimental.pallas import tpu as pltpu
from jax.experimental.pallas import tpu_sc as plsc

N_EMBD = 768
VOCAB = 50304
IBS = 256
PATCH_MAX = 10
KSIZE = 8
BEMB = N_EMBD // 2
B = 8
T = 256
END_TOK = VOCAB - 1
TM1 = T - 1
BPG = 4


_NW = 32
_ROWS = B * T
_RPW = _ROWS // _NW


def _sc_gather(table, idx_flat):
    mesh = plsc.VectorSubcoreMesh(core_axis_name="c", subcore_axis_name="s")

    @functools.partial(
        pl.kernel,
        out_type=jax.ShapeDtypeStruct((_ROWS, N_EMBD), jnp.float32),
        mesh=mesh,
        scratch_types=[
            pltpu.VMEM((_RPW,), jnp.int32),
            pltpu.VMEM((_RPW, N_EMBD), jnp.float32),
            pltpu.SemaphoreType.DMA,
        ],
    )
    def k(table_hbm, idx_hbm, out_hbm, idx_v, rows_v, sem):
        wid = lax.axis_index("s") * 2 + lax.axis_index("c")
        base = wid * _RPW
        pltpu.sync_copy(idx_hbm.at[pl.ds(base, _RPW)], idx_v)
        pltpu.async_copy(table_hbm.at[idx_v], rows_v, sem).wait()
        pltpu.sync_copy(rows_v, out_hbm.at[pl.ds(base, _RPW)])

    return k(table, idx_flat)


def _conv_body(x_ref, w_ref, loss_ref):
    xT = x_ref[0]
    p2 = jnp.zeros((TM1, BEMB), jnp.float32)
    for k in range(KSIZE):
        yk = lax.dot_general(xT, w_ref[k], (((1,), (0,)), ((), ())),
                             preferred_element_type=jnp.float32)
        off = KSIZE - 2 - k
        if off > 0:
            contrib = jnp.concatenate(
                [jnp.zeros((off, BEMB), jnp.float32), yk[: TM1 - off]], axis=0)
        elif off == 0:
            contrib = yk[:TM1]
        else:
            contrib = yk[1:T]
        p2 = p2 + contrib
    diff = xT[:TM1] - p2
    loss_ref[0] = jnp.mean(diff * diff, axis=1, keepdims=True)


def _conv_losses(tok_emb, w_kio):
    return pl.pallas_call(
        _conv_body,
        grid=(B,),
        in_specs=[
            pl.BlockSpec((1, T, BEMB), lambda b: (b, 0, 0)),
            pl.BlockSpec((KSIZE, BEMB, BEMB), lambda b: (0, 0, 0)),
        ],
        out_specs=pl.BlockSpec((1, TM1, 1), lambda b: (b, 0, 0)),
        out_shape=jax.ShapeDtypeStruct((B, TM1, 1), jnp.float32),
    )(tok_emb, w_kio)


def _rec_body(thr_ref, loss_ref, dep_ref, ln_ref):
    thr = thr_ref[0]

    def step(t, carry):
        acc, dep, ln = carry
        lv = loss_ref[pl.ds(t, 1), :]
        acc = acc + lv
        mask = (acc > thr) | (ln >= PATCH_MAX - 1)
        mi = mask.astype(jnp.int32)
        nmi = 1 - mi
        dep = dep + mi
        ln = (ln + nmi) * nmi
        acc = acc * nmi.astype(jnp.float32)
        dep_ref[pl.ds(t, 1), :] = dep
        ln_ref[pl.ds(t, 1), :] = ln
        return acc, dep, ln

    lax.fori_loop(0, TM1, step, (
        jnp.zeros((1, B), jnp.float32),
        jnp.zeros((1, B), jnp.int32),
        jnp.zeros((1, B), jnp.int32),
    ), unroll=4)


def _recurrence(losses_t, threshold):
    return pl.pallas_call(
        _rec_body,
        grid=(1,),
        in_specs=[
            pl.BlockSpec(memory_space=pltpu.SMEM),
            pl.BlockSpec((TM1, B), lambda i: (0, 0)),
        ],
        out_specs=[
            pl.BlockSpec((TM1, B), lambda i: (0, 0)),
            pl.BlockSpec((TM1, B), lambda i: (0, 0)),
        ],
        out_shape=[
            jax.ShapeDtypeStruct((TM1, B), jnp.int32),
            jax.ShapeDtypeStruct((TM1, B), jnp.int32),
        ],
    )(threshold, losses_t)


def _mlp_body(thr_ref, loss_ref, emb_ref, tid_col_ref, wpe_ref,
              w1_ref, b1_ref, w2_ref, b2_ref, out_ref, pi_ref,
              dep_sc, ln_sc):
    g = pl.program_id(0)
    ng = B // BPG

    @pl.when(g == 0)
    def _rec():
        dep_sc[...] = jnp.full((ng, T, BPG), -7, jnp.int32)
        ln_sc[...] = jnp.full((ng, T, BPG), -7, jnp.int32)
        thr = thr_ref[0]

        def step(t, carry):
            acc, dep, ln = carry
            lv = loss_ref[pl.ds(t, 1)].reshape(ng, BPG)
            acc = acc + lv
            mask = (acc > thr) | (ln >= PATCH_MAX - 1)
            mi = mask.astype(jnp.int32)
            nmi = 1 - mi
            dep = dep + mi
            ln = (ln + nmi) * nmi
            acc = acc * nmi.astype(jnp.float32)
            dep_sc[:, pl.ds(t, 1), :] = dep.reshape(ng, 1, BPG)
            ln_sc[:, pl.ds(t, 1), :] = ln.reshape(ng, 1, BPG)
            return acc, dep, ln

        lax.fori_loop(0, TM1, step, (
            jnp.zeros((ng, BPG), jnp.float32),
            jnp.zeros((ng, BPG), jnp.int32),
            jnp.zeros((ng, BPG), jnp.int32),
        ), unroll=4)

    R = BPG * T
    emb = emb_ref[...].reshape(R, N_EMBD)
    tid_col = tid_col_ref[...].reshape(R, 1)
    dep_g = dep_sc[pl.ds(g, 1)].reshape(T, BPG)
    ln_g = ln_sc[pl.ds(g, 1)].reshape(T, BPG)

    ys = [jnp.zeros((T, N_EMBD), jnp.float32) for _ in range(BPG)]
    posv = jnp.zeros((1, N_EMBD), jnp.float32)
    for s in range(PATCH_MAX):
        w1s = w1_ref[:, s * N_EMBD:(s + 1) * N_EMBD]
        ts = lax.dot_general(emb, w1s, (((1,), (1,)), ((), ())),
                             preferred_element_type=jnp.float32)
        for bi in range(BPG):
            mask_s = (ln_g[:, bi:bi + 1] == s).astype(jnp.float32)
            ys[bi] = ys[bi] + mask_s * ts[bi * T:(bi + 1) * T]
        posv = posv + lax.dot_general(wpe_ref[pl.ds(s, 1), :], w1s,
                                      (((1,), (1,)), ((), ())),
                                      preferred_element_type=jnp.float32)

    w2_bf = w2_ref[:].astype(jnp.bfloat16)
    p_row = lax.broadcasted_iota(jnp.int32, (1, T), 1)
    s_row = lax.broadcasted_iota(jnp.int32, (1, 16), 1)
    pb = posv + b1_ref[:]
    for bi in range(BPG):
        yb = ys[bi].astype(jnp.bfloat16)
        dep_col = dep_g[:, bi:bi + 1]
        at = (dep_col == p_row).astype(jnp.bfloat16)
        h = lax.dot_general(at, yb, (((0,), (0,)), ((), ())),
                            preferred_element_type=jnp.float32)
        h = h + pb
        h = 0.5 * h * (1.0 + lax.erf(h * 0.7071067811865476))
        out = lax.dot_general(h.astype(jnp.bfloat16), w2_bf,
                              (((1,), (1,)), ((), ())),
                              preferred_element_type=jnp.float32)
        out_ref[bi] = out + b2_ref[:]

        a2t = (dep_col == p_row + 1).astype(jnp.bfloat16)
        lnb = ln_g[:, bi:bi + 1]
        tid1 = tid_col[bi * T:(bi + 1) * T] + 1
        hit = (lnb == s_row)
        v_hi = jnp.where(hit, (tid1 >> 8), 0).astype(jnp.bfloat16)
        v_lo = jnp.where(hit, (tid1 & 255), 0).astype(jnp.bfloat16)
        v_fil = hit.astype(jnp.bfloat16)
        r_hi = lax.dot_general(a2t, v_hi, (((0,), (0,)), ((), ())),
                               preferred_element_type=jnp.float32)
        r_lo = lax.dot_general(a2t, v_lo, (((0,), (0,)), ((), ())),
                               preferred_element_type=jnp.float32)
        r_fil = lax.dot_general(a2t, v_fil, (((0,), (0,)), ((), ())),
                                preferred_element_type=jnp.float32)
        r_sel = r_hi * 256.0 + r_lo
        filled = r_fil > 0.5
        prev_fil = jnp.concatenate(
            [jnp.zeros((T, 1), jnp.float32), r_fil[:, :15]], axis=1) > 0.5
        pi = jnp.where(filled, r_sel - 1.0,
                       jnp.where(prev_fil, float(END_TOK), -1.0))
        pi_ref[bi] = pi[:, :PATCH_MAX].astype(jnp.int32)


def _mlp(threshold, losses_t, tok_emb, tid_col, wpe, w1, b1, w2, b2):
    ng = B // BPG
    return pl.pallas_call(
        _mlp_body,
        grid=(ng,),
        in_specs=[
            pl.BlockSpec(memory_space=pltpu.SMEM),
            pl.BlockSpec((TM1, ng, BPG), lambda b: (0, 0, 0)),
            pl.BlockSpec((BPG, T, N_EMBD), lambda b: (b, 0, 0)),
            pl.BlockSpec((BPG, T, 1), lambda b: (b, 0, 0)),
            pl.BlockSpec((PATCH_MAX, N_EMBD), lambda b: (0, 0)),
            pl.BlockSpec((N_EMBD, N_EMBD * PATCH_MAX), lambda b: (0, 0)),
            pl.BlockSpec((1, N_EMBD), lambda b: (0, 0)),
            pl.BlockSpec((N_EMBD, N_EMBD), lambda b: (0, 0)),
            pl.BlockSpec((1, N_EMBD), lambda b: (0, 0)),
        ],
        out_specs=[
            pl.BlockSpec((BPG, T, N_EMBD), lambda b: (b, 0, 0)),
            pl.BlockSpec((BPG, T, PATCH_MAX), lambda b: (b, 0, 0)),
        ],
        out_shape=[
            jax.ShapeDtypeStruct((B, T, N_EMBD), jnp.float32),
            jax.ShapeDtypeStruct((B, T, PATCH_MAX), jnp.int32),
        ],
        scratch_shapes=[
            pltpu.VMEM((ng, T, BPG), jnp.int32),
            pltpu.VMEM((ng, T, BPG), jnp.int32),
        ],
    )(threshold, losses_t, tok_emb, tid_col, wpe, w1, b1, w2, b2)


def kernel(idx, wte, wpe, conv_w, threshold, w1, b1, w2, b2):
    tok_flat = _sc_gather(wte, idx.reshape(-1))
    tok_emb = tok_flat.reshape(B, T, N_EMBD)

    w_kio = jnp.transpose(conv_w, (2, 1, 0))
    losses3 = _conv_losses(tok_emb, w_kio)
    losses = losses3.reshape(B, TM1)

    losses_t = jnp.transpose(losses).reshape(TM1, B // BPG, BPG)
    tid_col = idx.reshape(B, T, 1)

    out, pi = _mlp(threshold, losses_t, tok_emb, tid_col, wpe,
                   w1, b1.reshape(1, N_EMBD), w2, b2.reshape(1, N_EMBD))
    return out, pi, losses


# speedup vs baseline: 1.0290x; 1.0290x over previous
import functools

import jax
import jax.numpy as jnp
from jax import lax
from jax.experimental import pallas as pl
from jax.experimental.pallas import tpu as pltpu
from jax.experimental.pallas import tpu_sc as plsc

N_EMBD = 768
VOCAB = 50304
IBS = 256
PATCH_MAX = 10
KSIZE = 8
BEMB = N_EMBD // 2
B = 8
T = 256
END_TOK = VOCAB - 1
TM1 = T - 1
BPG = 4
CBG = 4


_NW = 32
_ROWS = B * T
_RPW = _ROWS // _NW


def _sc_gather(table, idx_flat):
    mesh = plsc.VectorSubcoreMesh(core_axis_name="c", subcore_axis_name="s")

    @functools.partial(
        pl.kernel,
        out_type=jax.ShapeDtypeStruct((_ROWS, N_EMBD), jnp.float32),
        mesh=mesh,
        scratch_types=[
            pltpu.VMEM((_RPW,), jnp.int32),
            pltpu.VMEM((_RPW, N_EMBD), jnp.float32),
            pltpu.SemaphoreType.DMA,
        ],
    )
    def k(table_hbm, idx_hbm, out_hbm, idx_v, rows_v, sem):
        wid = lax.axis_index("s") * 2 + lax.axis_index("c")
        base = wid * _RPW
        pltpu.sync_copy(idx_hbm.at[pl.ds(base, _RPW)], idx_v)
        pltpu.async_copy(table_hbm.at[idx_v], rows_v, sem).wait()
        pltpu.sync_copy(rows_v, out_hbm.at[pl.ds(base, _RPW)])

    return k(table, idx_flat)


def _conv_body(x_ref, w_ref, loss_ref, losst_ref):
    xf = x_ref[...].reshape(CBG * T, BEMB)
    yks = []
    for k in range(KSIZE):
        yks.append(lax.dot_general(xf, w_ref[k], (((1,), (0,)), ((), ())),
                                   preferred_element_type=jnp.float32))
    outs, cols = [], []
    for bi in range(CBG):
        base = bi * T
        p2 = jnp.zeros((TM1, BEMB), jnp.float32)
        for k in range(KSIZE):
            yk = yks[k]
            off = KSIZE - 2 - k
            if off > 0:
                contrib = jnp.concatenate(
                    [jnp.zeros((off, BEMB), jnp.float32),
                     yk[base:base + TM1 - off]], axis=0)
            elif off == 0:
                contrib = yk[base:base + TM1]
            else:
                contrib = yk[base + 1:base + T]
            p2 = p2 + contrib
        diff = xf[base:base + TM1] - p2
        l = jnp.mean(diff * diff, axis=1, keepdims=True)
        outs.append(l.reshape(1, TM1, 1))
        cols.append(l)
    loss_ref[...] = jnp.concatenate(outs, axis=0)
    losst_ref[...] = jnp.concatenate(cols, axis=1).reshape(1, TM1, CBG)


def _conv_losses(tok_emb, w_kio):
    return pl.pallas_call(
        _conv_body,
        grid=(B // CBG,),
        in_specs=[
            pl.BlockSpec((CBG, T, BEMB), lambda g: (g, 0, 0)),
            pl.BlockSpec((KSIZE, BEMB, BEMB), lambda g: (0, 0, 0)),
        ],
        out_specs=[
            pl.BlockSpec((CBG, TM1, 1), lambda g: (g, 0, 0)),
            pl.BlockSpec((1, TM1, CBG), lambda g: (g, 0, 0)),
        ],
        out_shape=[
            jax.ShapeDtypeStruct((B, TM1, 1), jnp.float32),
            jax.ShapeDtypeStruct((B // CBG, TM1, CBG), jnp.float32),
        ],
    )(tok_emb, w_kio)


def _rec_body(thr_ref, loss_ref, dep_ref, ln_ref):
    thr = thr_ref[0]

    def step(t, carry):
        acc, dep, ln = carry
        lv = loss_ref[pl.ds(t, 1), :]
        acc = acc + lv
        mask = (acc > thr) | (ln >= PATCH_MAX - 1)
        mi = mask.astype(jnp.int32)
        nmi = 1 - mi
        dep = dep + mi
        ln = (ln + nmi) * nmi
        acc = acc * nmi.astype(jnp.float32)
        dep_ref[pl.ds(t, 1), :] = dep
        ln_ref[pl.ds(t, 1), :] = ln
        return acc, dep, ln

    lax.fori_loop(0, TM1, step, (
        jnp.zeros((1, B), jnp.float32),
        jnp.zeros((1, B), jnp.int32),
        jnp.zeros((1, B), jnp.int32),
    ), unroll=4)


def _recurrence(losses_t, threshold):
    return pl.pallas_call(
        _rec_body,
        grid=(1,),
        in_specs=[
            pl.BlockSpec(memory_space=pltpu.SMEM),
            pl.BlockSpec((TM1, B), lambda i: (0, 0)),
        ],
        out_specs=[
            pl.BlockSpec((TM1, B), lambda i: (0, 0)),
            pl.BlockSpec((TM1, B), lambda i: (0, 0)),
        ],
        out_shape=[
            jax.ShapeDtypeStruct((TM1, B), jnp.int32),
            jax.ShapeDtypeStruct((TM1, B), jnp.int32),
        ],
    )(threshold, losses_t)


def _mlp_body(thr_ref, loss_ref, emb_ref, tid_col_ref, wpe_ref,
              w1_ref, b1_ref, w2_ref, b2_ref, out_ref, pi_ref,
              dep_sc, ln_sc):
    g = pl.program_id(0)
    ng = B // BPG

    @pl.when(g == 0)
    def _rec():
        dep_sc[...] = jnp.full((ng, T, BPG), -7, jnp.int32)
        ln_sc[...] = jnp.full((ng, T, BPG), -7, jnp.int32)
        thr = thr_ref[0]

        def step(t, carry):
            acc, dep, ln = carry
            lv = loss_ref[:, pl.ds(t, 1), :].reshape(ng, BPG)
            acc = acc + lv
            mask = (acc > thr) | (ln >= PATCH_MAX - 1)
            mi = mask.astype(jnp.int32)
            nmi = 1 - mi
            dep = dep + mi
            ln = (ln + nmi) * nmi
            acc = acc * nmi.astype(jnp.float32)
            dep_sc[:, pl.ds(t, 1), :] = dep.reshape(ng, 1, BPG)
            ln_sc[:, pl.ds(t, 1), :] = ln.reshape(ng, 1, BPG)
            return acc, dep, ln

        lax.fori_loop(0, TM1, step, (
            jnp.zeros((ng, BPG), jnp.float32),
            jnp.zeros((ng, BPG), jnp.int32),
            jnp.zeros((ng, BPG), jnp.int32),
        ), unroll=4)

    R = BPG * T
    emb = emb_ref[...].reshape(R, N_EMBD)
    tid_col = tid_col_ref[...].reshape(R, 1)
    dep_g = dep_sc[pl.ds(g, 1)].reshape(T, BPG)
    ln_g = ln_sc[pl.ds(g, 1)].reshape(T, BPG)

    ys = [jnp.zeros((T, N_EMBD), jnp.float32) for _ in range(BPG)]
    posv = jnp.zeros((1, N_EMBD), jnp.float32)
    for s in range(PATCH_MAX):
        w1s = w1_ref[:, s * N_EMBD:(s + 1) * N_EMBD]
        ts = lax.dot_general(emb, w1s, (((1,), (1,)), ((), ())),
                             preferred_element_type=jnp.float32)
        for bi in range(BPG):
            mask_s = (ln_g[:, bi:bi + 1] == s).astype(jnp.float32)
            ys[bi] = ys[bi] + mask_s * ts[bi * T:(bi + 1) * T]
        posv = posv + lax.dot_general(wpe_ref[pl.ds(s, 1), :], w1s,
                                      (((1,), (1,)), ((), ())),
                                      preferred_element_type=jnp.float32)

    w2_bf = w2_ref[:].astype(jnp.bfloat16)
    p_row = lax.broadcasted_iota(jnp.int32, (1, T), 1)
    s_row = lax.broadcasted_iota(jnp.int32, (1, 16), 1)
    pb = posv + b1_ref[:]
    for bi in range(BPG):
        yb = ys[bi].astype(jnp.bfloat16)
        dep_col = dep_g[:, bi:bi + 1]
        at = (dep_col == p_row).astype(jnp.bfloat16)
        h = lax.dot_general(at, yb, (((0,), (0,)), ((), ())),
                            preferred_element_type=jnp.float32)
        h = h + pb
        h = 0.5 * h * (1.0 + lax.erf(h * 0.7071067811865476))
        out = lax.dot_general(h.astype(jnp.bfloat16), w2_bf,
                              (((1,), (1,)), ((), ())),
                              preferred_element_type=jnp.float32)
        out_ref[bi] = out + b2_ref[:]

        a2t = (dep_col == p_row + 1).astype(jnp.bfloat16)
        lnb = ln_g[:, bi:bi + 1]
        tid1 = tid_col[bi * T:(bi + 1) * T] + 1
        hit = (lnb == s_row)
        v_hi = jnp.where(hit, (tid1 >> 8), 0).astype(jnp.bfloat16)
        v_lo = jnp.where(hit, (tid1 & 255), 0).astype(jnp.bfloat16)
        v_fil = hit.astype(jnp.bfloat16)
        r_hi = lax.dot_general(a2t, v_hi, (((0,), (0,)), ((), ())),
                               preferred_element_type=jnp.float32)
        r_lo = lax.dot_general(a2t, v_lo, (((0,), (0,)), ((), ())),
                               preferred_element_type=jnp.float32)
        r_fil = lax.dot_general(a2t, v_fil, (((0,), (0,)), ((), ())),
                                preferred_element_type=jnp.float32)
        r_sel = r_hi * 256.0 + r_lo
        filled = r_fil > 0.5
        prev_fil = jnp.concatenate(
            [jnp.zeros((T, 1), jnp.float32), r_fil[:, :15]], axis=1) > 0.5
        pi = jnp.where(filled, r_sel - 1.0,
                       jnp.where(prev_fil, float(END_TOK), -1.0))
        pi_ref[bi] = pi[:, :PATCH_MAX].astype(jnp.int32)


def _mlp(threshold, losses_t, tok_emb, tid_col, wpe, w1, b1, w2, b2):
    ng = B // BPG
    return pl.pallas_call(
        _mlp_body,
        grid=(ng,),
        in_specs=[
            pl.BlockSpec(memory_space=pltpu.SMEM),
            pl.BlockSpec((ng, TM1, BPG), lambda b: (0, 0, 0)),
            pl.BlockSpec((BPG, T, N_EMBD), lambda b: (b, 0, 0)),
            pl.BlockSpec((BPG, T, 1), lambda b: (b, 0, 0)),
            pl.BlockSpec((PATCH_MAX, N_EMBD), lambda b: (0, 0)),
            pl.BlockSpec((N_EMBD, N_EMBD * PATCH_MAX), lambda b: (0, 0)),
            pl.BlockSpec((1, N_EMBD), lambda b: (0, 0)),
            pl.BlockSpec((N_EMBD, N_EMBD), lambda b: (0, 0)),
            pl.BlockSpec((1, N_EMBD), lambda b: (0, 0)),
        ],
        out_specs=[
            pl.BlockSpec((BPG, T, N_EMBD), lambda b: (b, 0, 0)),
            pl.BlockSpec((BPG, T, PATCH_MAX), lambda b: (b, 0, 0)),
        ],
        out_shape=[
            jax.ShapeDtypeStruct((B, T, N_EMBD), jnp.float32),
            jax.ShapeDtypeStruct((B, T, PATCH_MAX), jnp.int32),
        ],
        scratch_shapes=[
            pltpu.VMEM((ng, T, BPG), jnp.int32),
            pltpu.VMEM((ng, T, BPG), jnp.int32),
        ],
    )(threshold, losses_t, tok_emb, tid_col, wpe, w1, b1, w2, b2)


def kernel(idx, wte, wpe, conv_w, threshold, w1, b1, w2, b2):
    tok_flat = _sc_gather(wte, idx.reshape(-1))
    tok_emb = tok_flat.reshape(B, T, N_EMBD)

    w_kio = jnp.transpose(conv_w, (2, 1, 0))
    losses3, losses_t = _conv_losses(tok_emb, w_kio)
    losses = losses3.reshape(B, TM1)
    tid_col = idx.reshape(B, T, 1)

    out, pi = _mlp(threshold, losses_t, tok_emb, tid_col, wpe,
                   w1, b1.reshape(1, N_EMBD), w2, b2.reshape(1, N_EMBD))
    return out, pi, losses


# speedup vs baseline: 1.0306x; 1.0015x over previous
import functools

import jax
import jax.numpy as jnp
from jax import lax
from jax.experimental import pallas as pl
from jax.experimental.pallas import tpu as pltpu
from jax.experimental.pallas import tpu_sc as plsc

N_EMBD = 768
VOCAB = 50304
IBS = 256
PATCH_MAX = 10
KSIZE = 8
BEMB = N_EMBD // 2
B = 8
T = 256
END_TOK = VOCAB - 1
TM1 = T - 1
BPG = 4
CBG = 4


_NW = 32
_ROWS = B * T
_RPW = _ROWS // _NW


def _sc_gather(table, idx_flat):
    mesh = plsc.VectorSubcoreMesh(core_axis_name="c", subcore_axis_name="s")

    @functools.partial(
        pl.kernel,
        out_type=jax.ShapeDtypeStruct((_ROWS, N_EMBD), jnp.float32),
        mesh=mesh,
        scratch_types=[
            pltpu.VMEM((_RPW,), jnp.int32),
            pltpu.VMEM((_RPW, N_EMBD), jnp.float32),
            pltpu.SemaphoreType.DMA,
        ],
    )
    def k(table_hbm, idx_hbm, out_hbm, idx_v, rows_v, sem):
        wid = lax.axis_index("s") * 2 + lax.axis_index("c")
        base = wid * _RPW
        pltpu.sync_copy(idx_hbm.at[pl.ds(base, _RPW)], idx_v)
        pltpu.async_copy(table_hbm.at[idx_v], rows_v, sem).wait()
        pltpu.sync_copy(rows_v, out_hbm.at[pl.ds(base, _RPW)])

    return k(table, idx_flat)


def _conv_body(x_ref, w_ref, loss_ref, losst_ref):
    xf = x_ref[...].reshape(CBG * T, BEMB)
    yks = []
    for k in range(KSIZE):
        yks.append(lax.dot_general(xf, w_ref[k], (((1,), (0,)), ((), ())),
                                   preferred_element_type=jnp.float32))
    outs, cols = [], []
    for bi in range(CBG):
        base = bi * T
        p2 = jnp.zeros((TM1, BEMB), jnp.float32)
        for k in range(KSIZE):
            yk = yks[k]
            off = KSIZE - 2 - k
            if off > 0:
                contrib = jnp.concatenate(
                    [jnp.zeros((off, BEMB), jnp.float32),
                     yk[base:base + TM1 - off]], axis=0)
            elif off == 0:
                contrib = yk[base:base + TM1]
            else:
                contrib = yk[base + 1:base + T]
            p2 = p2 + contrib
        diff = xf[base:base + TM1] - p2
        l = jnp.mean(diff * diff, axis=1, keepdims=True)
        outs.append(l.reshape(1, TM1, 1))
        cols.append(l)
    loss_ref[...] = jnp.concatenate(outs, axis=0)
    losst_ref[...] = jnp.concatenate(cols, axis=1).reshape(1, TM1, CBG)


def _conv_losses(tok_emb, w_kio):
    return pl.pallas_call(
        _conv_body,
        grid=(B // CBG,),
        in_specs=[
            pl.BlockSpec((CBG, T, BEMB), lambda g: (g, 0, 0)),
            pl.BlockSpec((KSIZE, BEMB, BEMB), lambda g: (0, 0, 0)),
        ],
        out_specs=[
            pl.BlockSpec((CBG, TM1, 1), lambda g: (g, 0, 0)),
            pl.BlockSpec((1, TM1, CBG), lambda g: (g, 0, 0)),
        ],
        out_shape=[
            jax.ShapeDtypeStruct((B, TM1, 1), jnp.float32),
            jax.ShapeDtypeStruct((B // CBG, TM1, CBG), jnp.float32),
        ],
    )(tok_emb, w_kio)


def _mlp_body(thr_ref, loss_ref, emb_ref, tid_col_ref, wpe_ref,
              w1_ref, b1_ref, w2_ref, b2_ref, out_ref, pi_ref,
              dep_sc, ln_sc):
    g = pl.program_id(0)
    ng = B // BPG

    @pl.when(g == 0)
    def _rec():
        dep_sc[...] = jnp.full((ng, T, BPG), -7, jnp.int32)
        ln_sc[...] = jnp.full((ng, T, BPG), -7, jnp.int32)
        thr = thr_ref[0]

        def step(t, carry):
            acc, dep, ln = carry
            lv = loss_ref[:, pl.ds(t, 1), :].reshape(ng, BPG)
            acc = acc + lv
            mask = (acc > thr) | (ln >= PATCH_MAX - 1)
            mi = mask.astype(jnp.int32)
            nmi = 1 - mi
            dep = dep + mi
            ln = (ln + nmi) * nmi
            acc = acc * nmi.astype(jnp.float32)
            dep_sc[:, pl.ds(t, 1), :] = dep.reshape(ng, 1, BPG)
            ln_sc[:, pl.ds(t, 1), :] = ln.reshape(ng, 1, BPG)
            return acc, dep, ln

        lax.fori_loop(0, TM1, step, (
            jnp.zeros((ng, BPG), jnp.float32),
            jnp.zeros((ng, BPG), jnp.int32),
            jnp.zeros((ng, BPG), jnp.int32),
        ), unroll=4)

    R = BPG * T
    emb = emb_ref[...].reshape(R, N_EMBD)
    tid_col = tid_col_ref[...].reshape(R, 1)
    dep_g = dep_sc[pl.ds(g, 1)].reshape(T, BPG)
    ln_g = ln_sc[pl.ds(g, 1)].reshape(T, BPG)

    ys = [jnp.zeros((T, N_EMBD), jnp.float32) for _ in range(BPG)]
    posv = jnp.zeros((1, N_EMBD), jnp.float32)
    for s in range(PATCH_MAX):
        w1s = w1_ref[:, s * N_EMBD:(s + 1) * N_EMBD]
        ts = lax.dot_general(emb, w1s, (((1,), (1,)), ((), ())),
                             preferred_element_type=jnp.float32)
        for bi in range(BPG):
            sel = ln_g[:, bi:bi + 1] == s
            ys[bi] = jnp.where(sel, ts[bi * T:(bi + 1) * T], ys[bi])
        posv = posv + lax.dot_general(wpe_ref[pl.ds(s, 1), :], w1s,
                                      (((1,), (1,)), ((), ())),
                                      preferred_element_type=jnp.float32)

    w2_bf = w2_ref[:].astype(jnp.bfloat16)
    p_row = lax.broadcasted_iota(jnp.int32, (1, T), 1)
    s_row = lax.broadcasted_iota(jnp.int32, (1, 16), 1)
    pb = posv + b1_ref[:]
    for bi in range(BPG):
        yb = ys[bi].astype(jnp.bfloat16)
        dep_col = dep_g[:, bi:bi + 1]
        at = (dep_col == p_row).astype(jnp.bfloat16)
        h = lax.dot_general(at, yb, (((0,), (0,)), ((), ())),
                            preferred_element_type=jnp.float32)
        h = h + pb
        h = 0.5 * h * (1.0 + lax.erf(h * 0.7071067811865476))
        out = lax.dot_general(h.astype(jnp.bfloat16), w2_bf,
                              (((1,), (1,)), ((), ())),
                              preferred_element_type=jnp.float32)
        out_ref[bi] = out + b2_ref[:]

        a2t = (dep_col == p_row + 1).astype(jnp.bfloat16)
        lnb = ln_g[:, bi:bi + 1]
        tid1 = tid_col[bi * T:(bi + 1) * T] + 1
        hit = (lnb == s_row)
        v_hi = jnp.where(hit, (tid1 >> 8), 0).astype(jnp.bfloat16)
        v_lo = jnp.where(hit, (tid1 & 255), 0).astype(jnp.bfloat16)
        v_fil = hit.astype(jnp.bfloat16)
        r_hi = lax.dot_general(a2t, v_hi, (((0,), (0,)), ((), ())),
                               preferred_element_type=jnp.float32)
        r_lo = lax.dot_general(a2t, v_lo, (((0,), (0,)), ((), ())),
                               preferred_element_type=jnp.float32)
        r_fil = lax.dot_general(a2t, v_fil, (((0,), (0,)), ((), ())),
                                preferred_element_type=jnp.float32)
        r_sel = r_hi * 256.0 + r_lo
        filled = r_fil > 0.5
        prev_fil = jnp.concatenate(
            [jnp.zeros((T, 1), jnp.float32), r_fil[:, :15]], axis=1) > 0.5
        pi = jnp.where(filled, r_sel - 1.0,
                       jnp.where(prev_fil, float(END_TOK), -1.0))
        pi_ref[bi] = pi[:, :PATCH_MAX].astype(jnp.int32)


def _mlp(threshold, losses_t, tok_emb, tid_col, wpe, w1, b1, w2, b2):
    ng = B // BPG
    return pl.pallas_call(
        _mlp_body,
        grid=(ng,),
        in_specs=[
            pl.BlockSpec(memory_space=pltpu.SMEM),
            pl.BlockSpec((ng, TM1, BPG), lambda b: (0, 0, 0)),
            pl.BlockSpec((BPG, T, N_EMBD), lambda b: (b, 0, 0)),
            pl.BlockSpec((BPG, T, 1), lambda b: (b, 0, 0)),
            pl.BlockSpec((PATCH_MAX, N_EMBD), lambda b: (0, 0)),
            pl.BlockSpec((N_EMBD, N_EMBD * PATCH_MAX), lambda b: (0, 0)),
            pl.BlockSpec((1, N_EMBD), lambda b: (0, 0)),
            pl.BlockSpec((N_EMBD, N_EMBD), lambda b: (0, 0)),
            pl.BlockSpec((1, N_EMBD), lambda b: (0, 0)),
        ],
        out_specs=[
            pl.BlockSpec((BPG, T, N_EMBD), lambda b: (b, 0, 0)),
            pl.BlockSpec((BPG, T, PATCH_MAX), lambda b: (b, 0, 0)),
        ],
        out_shape=[
            jax.ShapeDtypeStruct((B, T, N_EMBD), jnp.float32),
            jax.ShapeDtypeStruct((B, T, PATCH_MAX), jnp.int32),
        ],
        scratch_shapes=[
            pltpu.VMEM((ng, T, BPG), jnp.int32),
            pltpu.VMEM((ng, T, BPG), jnp.int32),
        ],
    )(threshold, losses_t, tok_emb, tid_col, wpe, w1, b1, w2, b2)


def kernel(idx, wte, wpe, conv_w, threshold, w1, b1, w2, b2):
    tok_flat = _sc_gather(wte, idx.reshape(-1))
    tok_emb = tok_flat.reshape(B, T, N_EMBD)

    w_kio = jnp.transpose(conv_w, (2, 1, 0))
    losses3, losses_t = _conv_losses(tok_emb, w_kio)
    losses = losses3.reshape(B, TM1)
    tid_col = idx.reshape(B, T, 1)

    out, pi = _mlp(threshold, losses_t, tok_emb, tid_col, wpe,
                   w1, b1.reshape(1, N_EMBD), w2, b2.reshape(1, N_EMBD))
    return out, pi, losses


# speedup vs baseline: 1.0434x; 1.0124x over previous
import functools

import jax
import jax.numpy as jnp
from jax import lax
from jax.experimental import pallas as pl
from jax.experimental.pallas import tpu as pltpu
from jax.experimental.pallas import tpu_sc as plsc

N_EMBD = 768
VOCAB = 50304
IBS = 256
PATCH_MAX = 10
KSIZE = 8
BEMB = N_EMBD // 2
B = 8
T = 256
END_TOK = VOCAB - 1
TM1 = T - 1
BPG = 4
CBG = 4


_NW = 32
_ROWS = B * T
_RPW = _ROWS // _NW
_HPW = _RPW // 2


def _sc_gather(table, idx_flat):
    mesh = plsc.VectorSubcoreMesh(core_axis_name="c", subcore_axis_name="s")

    @functools.partial(
        pl.kernel,
        out_type=jax.ShapeDtypeStruct((_ROWS, N_EMBD), jnp.float32),
        mesh=mesh,
        scratch_types=[
            pltpu.VMEM((_HPW,), jnp.int32),
            pltpu.VMEM((_HPW,), jnp.int32),
            pltpu.VMEM((_HPW, N_EMBD), jnp.float32),
            pltpu.VMEM((_HPW, N_EMBD), jnp.float32),
            pltpu.SemaphoreType.DMA,
            pltpu.SemaphoreType.DMA,
            pltpu.SemaphoreType.DMA,
        ],
    )
    def k(table_hbm, idx_hbm, out_hbm, idx_v0, idx_v1, rows0, rows1,
          sg0, sg1, sw):
        wid = lax.axis_index("s") * 2 + lax.axis_index("c")
        base = wid * _RPW
        pltpu.sync_copy(idx_hbm.at[pl.ds(base, _HPW)], idx_v0)
        g0 = pltpu.async_copy(table_hbm.at[idx_v0], rows0, sg0)
        pltpu.sync_copy(idx_hbm.at[pl.ds(base + _HPW, _HPW)], idx_v1)
        g1 = pltpu.async_copy(table_hbm.at[idx_v1], rows1, sg1)
        g0.wait()
        w0 = pltpu.async_copy(rows0, out_hbm.at[pl.ds(base, _HPW)], sw)
        g1.wait()
        w1 = pltpu.async_copy(rows1, out_hbm.at[pl.ds(base + _HPW, _HPW)], sw)
        w0.wait()
        w1.wait()

    return k(table, idx_flat)


def _conv_body(x_ref, w_ref, loss_ref, losst_ref):
    xf = x_ref[...].reshape(CBG * T, BEMB)
    yks = []
    for k in range(KSIZE):
        yks.append(lax.dot_general(xf, w_ref[k], (((1,), (0,)), ((), ())),
                                   preferred_element_type=jnp.float32))
    outs, cols = [], []
    for bi in range(CBG):
        base = bi * T
        p2 = jnp.zeros((TM1, BEMB), jnp.float32)
        for k in range(KSIZE):
            yk = yks[k]
            off = KSIZE - 2 - k
            if off > 0:
                contrib = jnp.concatenate(
                    [jnp.zeros((off, BEMB), jnp.float32),
                     yk[base:base + TM1 - off]], axis=0)
            elif off == 0:
                contrib = yk[base:base + TM1]
            else:
                contrib = yk[base + 1:base + T]
            p2 = p2 + contrib
        diff = xf[base:base + TM1] - p2
        l = jnp.mean(diff * diff, axis=1, keepdims=True)
        outs.append(l.reshape(1, TM1, 1))
        cols.append(l)
    loss_ref[...] = jnp.concatenate(outs, axis=0)
    losst_ref[...] = jnp.concatenate(cols, axis=1).reshape(1, TM1, CBG)


def _conv_losses(tok_emb, w_kio):
    return pl.pallas_call(
        _conv_body,
        grid=(B // CBG,),
        in_specs=[
            pl.BlockSpec((CBG, T, BEMB), lambda g: (g, 0, 0)),
            pl.BlockSpec((KSIZE, BEMB, BEMB), lambda g: (0, 0, 0)),
        ],
        out_specs=[
            pl.BlockSpec((CBG, TM1, 1), lambda g: (g, 0, 0)),
            pl.BlockSpec((1, TM1, CBG), lambda g: (g, 0, 0)),
        ],
        out_shape=[
            jax.ShapeDtypeStruct((B, TM1, 1), jnp.float32),
            jax.ShapeDtypeStruct((B // CBG, TM1, CBG), jnp.float32),
        ],
    )(tok_emb, w_kio)


def _mlp_body(thr_ref, loss_ref, emb_ref, tid_col_ref, wpe_ref,
              w1_ref, b1_ref, w2_ref, b2_ref, out_ref, pi_ref,
              dep_sc, ln_sc):
    g = pl.program_id(0)
    ng = B // BPG

    @pl.when(g == 0)
    def _rec():
        dep_sc[...] = jnp.full((ng, T, BPG), -7, jnp.int32)
        ln_sc[...] = jnp.full((ng, T, BPG), -7, jnp.int32)
        thr = thr_ref[0]

        def step(t, carry):
            acc, dep, ln = carry
            lv = loss_ref[:, pl.ds(t, 1), :].reshape(ng, BPG)
            acc = acc + lv
            mask = (acc > thr) | (ln >= PATCH_MAX - 1)
            mi = mask.astype(jnp.int32)
            nmi = 1 - mi
            dep = dep + mi
            ln = (ln + nmi) * nmi
            acc = acc * nmi.astype(jnp.float32)
            dep_sc[:, pl.ds(t, 1), :] = dep.reshape(ng, 1, BPG)
            ln_sc[:, pl.ds(t, 1), :] = ln.reshape(ng, 1, BPG)
            return acc, dep, ln

        lax.fori_loop(0, TM1, step, (
            jnp.zeros((ng, BPG), jnp.float32),
            jnp.zeros((ng, BPG), jnp.int32),
            jnp.zeros((ng, BPG), jnp.int32),
        ), unroll=4)

    R = BPG * T
    emb = emb_ref[...].reshape(R, N_EMBD)
    tid_col = tid_col_ref[...].reshape(R, 1)
    dep_g = dep_sc[pl.ds(g, 1)].reshape(T, BPG)
    ln_g = ln_sc[pl.ds(g, 1)].reshape(T, BPG)

    ys = [jnp.zeros((T, N_EMBD), jnp.float32) for _ in range(BPG)]
    posv = jnp.zeros((1, N_EMBD), jnp.float32)
    for s in range(PATCH_MAX):
        w1s = w1_ref[:, s * N_EMBD:(s + 1) * N_EMBD]
        ts = lax.dot_general(emb, w1s, (((1,), (1,)), ((), ())),
                             preferred_element_type=jnp.float32)
        for bi in range(BPG):
            sel = ln_g[:, bi:bi + 1] == s
            ys[bi] = jnp.where(sel, ts[bi * T:(bi + 1) * T], ys[bi])
        posv = posv + lax.dot_general(wpe_ref[pl.ds(s, 1), :], w1s,
                                      (((1,), (1,)), ((), ())),
                                      preferred_element_type=jnp.float32)

    w2_bf = w2_ref[:].astype(jnp.bfloat16)
    p_row = lax.broadcasted_iota(jnp.int32, (1, T), 1)
    s_row = lax.broadcasted_iota(jnp.int32, (1, 16), 1)
    pb = posv + b1_ref[:]
    for bi in range(BPG):
        yb = ys[bi].astype(jnp.bfloat16)
        dep_col = dep_g[:, bi:bi + 1]
        at = (dep_col == p_row).astype(jnp.bfloat16)
        h = lax.dot_general(at, yb, (((0,), (0,)), ((), ())),
                            preferred_element_type=jnp.float32)
        h = h + pb
        h = 0.5 * h * (1.0 + lax.erf(h * 0.7071067811865476))
        out = lax.dot_general(h.astype(jnp.bfloat16), w2_bf,
                              (((1,), (1,)), ((), ())),
                              preferred_element_type=jnp.float32)
        out_ref[bi] = out + b2_ref[:]

        a2t = (dep_col == p_row + 1).astype(jnp.bfloat16)
        lnb = ln_g[:, bi:bi + 1]
        tid1 = tid_col[bi * T:(bi + 1) * T] + 1
        hit = (lnb == s_row)
        v_hi = jnp.where(hit, (tid1 >> 8), 0).astype(jnp.bfloat16)
        v_lo = jnp.where(hit, (tid1 & 255), 0).astype(jnp.bfloat16)
        v_fil = hit.astype(jnp.bfloat16)
        r_hi = lax.dot_general(a2t, v_hi, (((0,), (0,)), ((), ())),
                               preferred_element_type=jnp.float32)
        r_lo = lax.dot_general(a2t, v_lo, (((0,), (0,)), ((), ())),
                               preferred_element_type=jnp.float32)
        r_fil = lax.dot_general(a2t, v_fil, (((0,), (0,)), ((), ())),
                                preferred_element_type=jnp.float32)
        r_sel = r_hi * 256.0 + r_lo
        filled = r_fil > 0.5
        prev_fil = jnp.concatenate(
            [jnp.zeros((T, 1), jnp.float32), r_fil[:, :15]], axis=1) > 0.5
        pi = jnp.where(filled, r_sel - 1.0,
                       jnp.where(prev_fil, float(END_TOK), -1.0))
        pi_ref[bi] = pi[:, :PATCH_MAX].astype(jnp.int32)


def _mlp(threshold, losses_t, tok_emb, tid_col, wpe, w1, b1, w2, b2):
    ng = B // BPG
    return pl.pallas_call(
        _mlp_body,
        grid=(ng,),
        in_specs=[
            pl.BlockSpec(memory_space=pltpu.SMEM),
            pl.BlockSpec((ng, TM1, BPG), lambda b: (0, 0, 0)),
            pl.BlockSpec((BPG, T, N_EMBD), lambda b: (b, 0, 0)),
            pl.BlockSpec((BPG, T, 1), lambda b: (b, 0, 0)),
            pl.BlockSpec((PATCH_MAX, N_EMBD), lambda b: (0, 0)),
            pl.BlockSpec((N_EMBD, N_EMBD * PATCH_MAX), lambda b: (0, 0)),
            pl.BlockSpec((1, N_EMBD), lambda b: (0, 0)),
            pl.BlockSpec((N_EMBD, N_EMBD), lambda b: (0, 0)),
            pl.BlockSpec((1, N_EMBD), lambda b: (0, 0)),
        ],
        out_specs=[
            pl.BlockSpec((BPG, T, N_EMBD), lambda b: (b, 0, 0)),
            pl.BlockSpec((BPG, T, PATCH_MAX), lambda b: (b, 0, 0)),
        ],
        out_shape=[
            jax.ShapeDtypeStruct((B, T, N_EMBD), jnp.float32),
            jax.ShapeDtypeStruct((B, T, PATCH_MAX), jnp.int32),
        ],
        scratch_shapes=[
            pltpu.VMEM((ng, T, BPG), jnp.int32),
            pltpu.VMEM((ng, T, BPG), jnp.int32),
        ],
    )(threshold, losses_t, tok_emb, tid_col, wpe, w1, b1, w2, b2)


def kernel(idx, wte, wpe, conv_w, threshold, w1, b1, w2, b2):
    tok_flat = _sc_gather(wte, idx.reshape(-1))
    tok_emb = tok_flat.reshape(B, T, N_EMBD)

    w_kio = jnp.transpose(conv_w, (2, 1, 0))
    losses3, losses_t = _conv_losses(tok_emb, w_kio)
    losses = losses3.reshape(B, TM1)
    tid_col = idx.reshape(B, T, 1)

    out, pi = _mlp(threshold, losses_t, tok_emb, tid_col, wpe,
                   w1, b1.reshape(1, N_EMBD), w2, b2.reshape(1, N_EMBD))
    return out, pi, losses
